# Initial kernel scaffold; baseline (speedup 1.0000x reference)
import jax
import jax.numpy as jnp
from jax.experimental import pallas as pl


def kernel(temporal, spatial_prompt, emb0, emb1, emb2, emb3, emb4, emb5):
    raise NotImplementedError("write your pallas kernel here")


# speedup vs baseline: 1.6547x; 1.6547x over previous
import functools

import jax
import jax.numpy as jnp
from jax import lax
from jax.experimental import pallas as pl
from jax.experimental.pallas import tpu as pltpu
from jax.experimental.pallas import tpu_sc as plsc

DENORM = (1440, 24, 31, 53, 7, 12)
OFFS = (0, 1440, 1464, 1495, 1548, 1555)
SPATIAL_OFF = 1567
FEATURE_DIM = 3
NODE = 10000
MD = 64
BATCH = 16
ROWS = BATCH * NODE
LANES = 16
CHUNK = 128
NCHUNKS = ROWS // CHUNK
NC, NS = 2, 16
NW = NC * NS
CPW = (NCHUNKS + NW - 1) // NW
NT = 7


def _body(vals_hbm, table_hbm, out_hbm,
          vals_v, i0, i1, i2, i3, i4, i5, i6,
          r0, r1, r2, r3, r4, r5, r6, acc_v, sem):
    idx_refs = (i0, i1, i2, i3, i4, i5, i6)
    row_refs = (r0, r1, r2, r3, r4, r5, r6)
    wid = lax.axis_index("s") * NC + lax.axis_index("c")

    def chunk_body(k, _):
        t = k * NW + wid

        @pl.when(t < NCHUNKS)
        def _():
            row0 = t * CHUNK
            for i in range(6):
                pltpu.sync_copy(vals_hbm.at[i, pl.ds(row0, CHUNK)],
                                vals_v.at[i])
            for v in range(CHUNK // LANES):
                sl = pl.ds(v * LANES, LANES)
                for i in range(6):
                    x = vals_v[i, sl]
                    idx_refs[i][sl] = (x * DENORM[i]).astype(jnp.int32) + OFFS[i]
                r = row0 + v * LANES + lax.iota(jnp.int32, LANES)
                idx_refs[6][sl] = lax.rem(r, NODE) + SPATIAL_OFF
            cps = [pltpu.async_copy(table_hbm.at[idx_refs[j]], row_refs[j], sem)
                   for j in range(NT)]
            for c in cps:
                c.wait()

            def sum_row(i, _):
                for c4 in range(MD // LANES):
                    sl = pl.ds(c4 * LANES, LANES)
                    a = row_refs[0][i, sl]
                    for j in range(1, NT):
                        a = a + row_refs[j][i, sl]
                    acc_v[i, sl] = a
                return _

            lax.fori_loop(0, CHUNK, sum_row, None)
            pltpu.sync_copy(acc_v, out_hbm.at[pl.ds(row0, CHUNK)])
        return _

    lax.fori_loop(0, CPW, chunk_body, None)


@jax.jit
def kernel(temporal, spatial_prompt, emb0, emb1, emb2, emb3, emb4, emb5):
    vals = temporal[:, -1, :, FEATURE_DIM:FEATURE_DIM + 6]
    vals_t = vals.reshape(ROWS, 6).T
    table = jnp.concatenate(
        [emb0, emb1, emb2, emb3, emb4, emb5, spatial_prompt], axis=0)

    mesh = plsc.VectorSubcoreMesh(core_axis_name="c", subcore_axis_name="s",
                                  num_cores=NC, num_subcores=NS)
    scratch = (
        [pltpu.VMEM((6, CHUNK), jnp.float32)]
        + [pltpu.VMEM((CHUNK,), jnp.int32) for _ in range(NT)]
        + [pltpu.VMEM((CHUNK, MD), jnp.float32) for _ in range(NT)]
        + [pltpu.VMEM((CHUNK, MD), jnp.float32),
           pltpu.SemaphoreType.DMA]
    )
    out = pl.kernel(
        _body,
        out_type=jax.ShapeDtypeStruct((ROWS, MD), jnp.float32),
        mesh=mesh,
        scratch_types=scratch,
        compiler_params=pltpu.CompilerParams(use_tc_tiling_on_sc=False),
    )(vals_t, table)
    return out.reshape(BATCH, NODE, MD)


# speedup vs baseline: 1.6660x; 1.0068x over previous
import functools

import jax
import jax.numpy as jnp
from jax import lax
from jax.experimental import pallas as pl
from jax.experimental.pallas import tpu as pltpu
from jax.experimental.pallas import tpu_sc as plsc

DENORM = (1440, 24, 31, 53, 7, 12)
OFFS = (0, 1440, 1464, 1495, 1548, 1555)
SPATIAL_OFF = 1567
FEATURE_DIM = 3
NODE = 10000
MD = 64
BATCH = 16
ROWS = BATCH * NODE
LANES = 16
CHUNK = 128
NCHUNKS = ROWS // CHUNK
NC, NS = 2, 16
NW = NC * NS
CPW = (NCHUNKS + NW - 1) // NW
NT = 7


def _body(vals_hbm, table_hbm, out_hbm, vals_v, idx_v, acc_v, sem, sem_add):
    wid = lax.axis_index("s") * NC + lax.axis_index("c")

    def chunk_body(k, _):
        t = k * NW + wid

        @pl.when(t < NCHUNKS)
        def _():
            row0 = t * CHUNK
            pltpu.sync_copy(vals_hbm.at[:, pl.ds(row0, CHUNK)], vals_v)
            for v in range(CHUNK // LANES):
                sl = pl.ds(v * LANES, LANES)
                for i in range(6):
                    x = vals_v[i, sl]
                    idx_v[i, sl] = (x * DENORM[i]).astype(jnp.int32) + OFFS[i]
                r = row0 + v * LANES + lax.iota(jnp.int32, LANES)
                idx_v[6, sl] = lax.rem(r, NODE) + SPATIAL_OFF
            pltpu.async_copy(table_hbm.at[idx_v.at[6]], acc_v, sem).wait()
            cps = [pltpu.async_copy(table_hbm.at[idx_v.at[j]], acc_v,
                                    sem_add, add=True)
                   for j in range(6)]
            for c in cps:
                c.wait()
            pltpu.sync_copy(acc_v, out_hbm.at[pl.ds(row0, CHUNK)])
        return _

    lax.fori_loop(0, CPW, chunk_body, None)


@jax.jit
def kernel(temporal, spatial_prompt, emb0, emb1, emb2, emb3, emb4, emb5):
    vals = temporal[:, -1, :, FEATURE_DIM:FEATURE_DIM + 6]
    vals_t = vals.reshape(ROWS, 6).T
    table = jnp.concatenate(
        [emb0, emb1, emb2, emb3, emb4, emb5, spatial_prompt], axis=0)

    mesh = plsc.VectorSubcoreMesh(core_axis_name="c", subcore_axis_name="s",
                                  num_cores=NC, num_subcores=NS)
    scratch = (
        pltpu.VMEM((6, CHUNK), jnp.float32),
        pltpu.VMEM((NT, CHUNK), jnp.int32),
        pltpu.VMEM((CHUNK, MD), jnp.float32),
        pltpu.SemaphoreType.DMA,
        pltpu.SemaphoreType.DMA,
    )
    out = pl.kernel(
        _body,
        out_type=jax.ShapeDtypeStruct((ROWS, MD), jnp.float32),
        mesh=mesh,
        scratch_types=scratch,
        compiler_params=pltpu.CompilerParams(use_tc_tiling_on_sc=False),
    )(vals_t, table)
    return out.reshape(BATCH, NODE, MD)
